# baseline (speedup 1.0000x reference)
import functools

import jax
import jax.numpy as jnp
from jax import lax
from jax.experimental import pallas as pl
from jax.experimental.pallas import tpu as pltpu
from jax.experimental.pallas import tpu_sc as plsc

N = 10000
D = 128
E = 320000
G = 128
NW = 32
K = 128
NB = 80
EPW = NB * K
E_PAD = NW * EPW
NPAD = 10240
RPS = NPAD // 16
RB = 2000

_MESH = plsc.VectorSubcoreMesh(core_axis_name="c", subcore_axis_name="s")
_MESH1 = plsc.VectorSubcoreMesh(core_axis_name="c", subcore_axis_name="s",
                                num_cores=1)


def _sc_degree(dst3):
    @functools.partial(
        pl.kernel,
        out_type=jax.ShapeDtypeStruct((2, NPAD), jnp.float32),
        mesh=_MESH,
        scratch_types=[
            pltpu.VMEM((NB, K), jnp.int32),
            pltpu.VMEM((RPS,), jnp.float32),
            pltpu.VMEM((K,), jnp.float32),
            pltpu.VMEM_SHARED((NPAD,), jnp.float32),
        ],
    )
    def deg_kernel(dst_hbm, out_hbm, dst_v, zeros_v, ones_v, acc):
        cid = lax.axis_index("c")
        sid = lax.axis_index("s")
        wid = sid * 2 + cid

        @pl.loop(0, RPS, step=16)
        def _(i):
            zeros_v.at[pl.ds(i, 16)][...] = jnp.zeros((16,), jnp.float32)

        @pl.loop(0, K, step=16)
        def _(i):
            ones_v.at[pl.ds(i, 16)][...] = jnp.full((16,), 1.0, jnp.float32)

        pltpu.sync_copy(zeros_v, acc.at[pl.ds(sid * RPS, RPS)])
        pltpu.sync_copy(dst_hbm.at[wid], dst_v)
        plsc.subcore_barrier()

        @pl.loop(0, NB)
        def _(j):
            pltpu.sync_copy(ones_v, acc.at[dst_v.at[j]], add=True)

        plsc.subcore_barrier()
        pltpu.sync_copy(acc.at[pl.ds(sid * RPS, RPS)],
                        out_hbm.at[cid, pl.ds(sid * RPS, RPS)])

    return deg_kernel(dst3)


NBUF = 2
NI = 2 * NBUF

B0 = 156
B1 = 4
NBLK = 16 * (B0 + B1)
NB_MAX = ((max(B0, B1) + NI - 1) // NI) * NI


def _sc_spmm(scaled, sd3):
    @functools.partial(
        pl.kernel,
        out_type=jax.ShapeDtypeStruct((NPAD, D), jnp.float32),
        mesh=_MESH1,
        scratch_types=(
            [pltpu.VMEM((2, K), jnp.int32) for _ in range(NI)]
            + [pltpu.VMEM((K, D), jnp.float32) for _ in range(NBUF)]
            + [pltpu.SemaphoreType.DMA for _ in range(NI + NBUF)]
            + [pltpu.VMEM_SHARED((NPAD, D), jnp.float32)]
        ),
    )
    def spmm_kernel(scaled_hbm, sd_hbm, out_hbm, *rest):
        sd = rest[:NI]
        rows = rest[NI:NI + NBUF]
        sem_i = rest[NI + NBUF:2 * NI + NBUF]
        sem_g = rest[2 * NI + NBUF:2 * NI + 2 * NBUF]
        acc = rest[2 * NI + 2 * NBUF]
        sid = lax.axis_index("s")
        nb = jnp.int32(160)
        base = sid * 160

        @pl.loop(0, K)
        def _(r):
            @pl.loop(0, D, step=16)
            def _(cc):
                rows[0].at[r, pl.ds(cc, 16)][...] = jnp.zeros((16,), jnp.float32)

        @pl.loop(0, RPS, step=K)
        def _(r):
            pltpu.sync_copy(rows[0], acc.at[pl.ds(sid * RPS + r, K)])

        for q in range(NI):
            @pl.when(q < nb)
            def _():
                pltpu.async_copy(sd_hbm.at[base + q], sd[q], sem_i[q])
        plsc.subcore_barrier()
        for b in range(NBUF):
            @pl.when(b < nb)
            def _():
                pltpu.make_async_copy(sd_hbm.at[base + b], sd[b],
                                      sem_i[b]).wait()
                pltpu.async_copy(scaled_hbm.at[sd[b].at[0]], rows[b], sem_g[b])

        nb_up = ((nb + NI - 1) // NI) * NI

        @pl.loop(0, nb_up, step=NI)
        def _(j0):
            for q in range(NI):
                j = j0 + q
                b = q % NBUF

                @pl.when(j < nb)
                def _():
                    pltpu.make_async_copy(scaled_hbm.at[sd[q].at[0]], rows[b],
                                          sem_g[b]).wait()
                    pltpu.sync_copy(rows[b], acc.at[sd[q].at[1]], add=True)

                    @pl.when(j + NBUF < nb)
                    def _():
                        qq = (q + NBUF) % NI
                        pltpu.make_async_copy(sd_hbm.at[base + j + NBUF],
                                              sd[qq], sem_i[qq]).wait()
                        pltpu.async_copy(scaled_hbm.at[sd[qq].at[0]], rows[b],
                                         sem_g[b])

                    @pl.when(j + NI < nb)
                    def _():
                        pltpu.async_copy(sd_hbm.at[base + j + NI], sd[q],
                                         sem_i[q])

        plsc.subcore_barrier()
        pltpu.sync_copy(acc.at[pl.ds(sid * RPS, RPS)],
                        out_hbm.at[pl.ds(sid * RPS, RPS)])

    return spmm_kernel(scaled, sd3)


def _enc_body(x_ref, w1_ref, b1_ref, w2_ref, b2_ref, h_ref):
    h1 = jnp.dot(x_ref[...], w1_ref[...], preferred_element_type=jnp.float32)
    h1 = jnp.maximum(h1 + b1_ref[...], 0.0)
    h2 = jnp.dot(h1, w2_ref[...], preferred_element_type=jnp.float32)
    h_ref[...] = jnp.maximum(h2 + b2_ref[...], 0.0)


def _encoder(x, W1, b1, W2, b2):
    return pl.pallas_call(
        _enc_body,
        grid=(N // RB,),
        in_specs=[
            pl.BlockSpec((RB, D), lambda i: (i, 0)),
            pl.BlockSpec((D, D), lambda i: (0, 0)),
            pl.BlockSpec((1, D), lambda i: (0, 0)),
            pl.BlockSpec((D, D), lambda i: (0, 0)),
            pl.BlockSpec((1, D), lambda i: (0, 0)),
        ],
        out_specs=pl.BlockSpec((RB, D), lambda i: (i, 0)),
        out_shape=jax.ShapeDtypeStruct((N, D), jnp.float32),
    )(x, W1, b1, W2, b2)


def _scale_body(degp_ref, h_ref, dinv_ref, scaled_ref):
    dp = degp_ref[...]
    deg = dp[0] + dp[1]
    dinv = lax.rsqrt(jnp.maximum(deg, 1.0))
    dinv_ref[...] = dinv
    scaled_ref[...] = h_ref[...] * dinv


def _scale(degp, h):
    return pl.pallas_call(
        _scale_body,
        grid=(N // RB,),
        in_specs=[
            pl.BlockSpec((2, RB, 1), lambda i: (0, i, 0)),
            pl.BlockSpec((RB, D), lambda i: (i, 0)),
        ],
        out_specs=[
            pl.BlockSpec((RB, 1), lambda i: (i, 0)),
            pl.BlockSpec((RB, D), lambda i: (i, 0)),
        ],
        out_shape=[
            jax.ShapeDtypeStruct((N, 1), jnp.float32),
            jax.ShapeDtypeStruct((N, D), jnp.float32),
        ],
    )(degp, h)


def _lap_body(aggp_ref, feat_ref, dinv_ref, u_ref, scaled_ref):
    ap = aggp_ref[...]
    dinv = dinv_ref[...]
    u = feat_ref[...] - dinv * ap
    u_ref[...] = u
    scaled_ref[...] = dinv * u


def _lap(aggp, feat, dinv):
    return pl.pallas_call(
        _lap_body,
        grid=(N // RB,),
        in_specs=[
            pl.BlockSpec((RB, D), lambda i: (i, 0)),
            pl.BlockSpec((RB, D), lambda i: (i, 0)),
            pl.BlockSpec((RB, 1), lambda i: (i, 0)),
        ],
        out_specs=[
            pl.BlockSpec((RB, D), lambda i: (i, 0)),
            pl.BlockSpec((RB, D), lambda i: (i, 0)),
        ],
        out_shape=[
            jax.ShapeDtypeStruct((N, D), jnp.float32),
            jax.ShapeDtypeStruct((N, D), jnp.float32),
        ],
    )(aggp, feat, dinv)


def _final_body(u1_ref, aggp_ref, h_ref, dinv_ref, batch_ref,
                w3_ref, b3_ref, w4_ref, b4_ref, out_ref,
                sums_ref, counts_ref):
    i = pl.program_id(0)
    ap = aggp_ref[...]
    u1 = u1_ref[...]
    u2 = u1 - dinv_ref[...] * ap
    h = h_ref[...]
    W3 = w3_ref[...]
    Wh = 3.0 * W3[0:D]
    Wa = -3.0 * W3[0:D] + 3.0 * W3[D:2 * D]
    Wb = 0.75 * W3[0:D] - 1.5 * W3[D:2 * D] + 0.75 * W3[2 * D:3 * D]
    y = (jnp.dot(h, Wh, preferred_element_type=jnp.float32)
         + jnp.dot(u1, Wa, preferred_element_type=jnp.float32)
         + jnp.dot(u2, Wb, preferred_element_type=jnp.float32)
         + b3_ref[...])
    y = jnp.maximum(y, 0.0)
    node = jnp.dot(y, w4_ref[...], preferred_element_type=jnp.float32)
    node = node + b4_ref[...]

    gids = lax.broadcasted_iota(jnp.int32, (1, G), 1)
    onehot = (batch_ref[...] == gids).astype(jnp.float32)
    psum = lax.dot_general(onehot, node, (((0,), (0,)), ((), ())),
                           preferred_element_type=jnp.float32)
    pcnt = lax.dot_general(onehot, jnp.ones((onehot.shape[0], 1), jnp.float32),
                           (((0,), (0,)), ((), ())),
                           preferred_element_type=jnp.float32)

    @pl.when(i == 0)
    def _():
        sums_ref[...] = jnp.zeros_like(sums_ref)
        counts_ref[...] = jnp.zeros_like(counts_ref)

    sums_ref[...] += psum
    counts_ref[...] += pcnt

    @pl.when(i == pl.num_programs(0) - 1)
    def _():
        out_ref[...] = sums_ref[...] / jnp.maximum(counts_ref[...], 1.0)


def _final(u1, aggp, h, dinv, batch2, W3, b3, W4, b4):
    return pl.pallas_call(
        _final_body,
        grid=(N // RB,),
        in_specs=[
            pl.BlockSpec((RB, D), lambda i: (i, 0)),
            pl.BlockSpec((RB, D), lambda i: (i, 0)),
            pl.BlockSpec((RB, D), lambda i: (i, 0)),
            pl.BlockSpec((RB, 1), lambda i: (i, 0)),
            pl.BlockSpec((RB, 1), lambda i: (i, 0)),
            pl.BlockSpec((3 * D, D), lambda i: (0, 0)),
            pl.BlockSpec((1, D), lambda i: (0, 0)),
            pl.BlockSpec((D, D), lambda i: (0, 0)),
            pl.BlockSpec((1, D), lambda i: (0, 0)),
        ],
        out_specs=pl.BlockSpec((G, D), lambda i: (0, 0)),
        out_shape=jax.ShapeDtypeStruct((G, D), jnp.float32),
        scratch_shapes=[
            pltpu.VMEM((G, D), jnp.float32),
            pltpu.VMEM((G, 1), jnp.float32),
        ],
    )(u1, aggp, h, dinv, batch2, W3, b3, W4, b4)


def kernel(x, edge_index, batch, W1, b1, W2, b2, W3, b3, W4, b4):
    src = edge_index[0]
    dst = edge_index[1]
    pad = E_PAD - E
    src_p = jnp.concatenate([src, jnp.zeros((pad,), jnp.int32)])
    dst_p = jnp.concatenate([dst, jnp.full((pad,), N, jnp.int32)])
    dst3 = dst_p.reshape(NW, NB, K)
    sd3 = jnp.stack([src_p.reshape(NBLK, K), dst_p.reshape(NBLK, K)],
                    axis=1)

    h = _encoder(x, W1, b1.reshape(1, D), W2, b2.reshape(1, D))
    degp = _sc_degree(dst3)
    dinv, scaled1 = _scale(degp.reshape(2, NPAD, 1), h)
    agg1 = _sc_spmm(scaled1, sd3)
    u1, scaled2 = _lap(agg1, h, dinv)
    agg2 = _sc_spmm(scaled2, sd3)
    return _final(u1, agg2, h, dinv, batch.reshape(N, 1),
                  W3, b3.reshape(1, D), W4, b4.reshape(1, D))


# speedup vs baseline: 1.1540x; 1.1540x over previous
import functools

import jax
import jax.numpy as jnp
from jax import lax
from jax.experimental import pallas as pl
from jax.experimental.pallas import tpu as pltpu
from jax.experimental.pallas import tpu_sc as plsc

N = 10000
D = 128
E = 320000
G = 128
NW = 32
K = 128
NB = 80
EPW = NB * K
E_PAD = NW * EPW
NPAD = 10240
RPS = NPAD // 16
RB = 2000

_MESH = plsc.VectorSubcoreMesh(core_axis_name="c", subcore_axis_name="s")
_MESH1 = plsc.VectorSubcoreMesh(core_axis_name="c", subcore_axis_name="s",
                                num_cores=1)


def _sc_degree(dst3):
    @functools.partial(
        pl.kernel,
        out_type=jax.ShapeDtypeStruct((2, NPAD), jnp.float32),
        mesh=_MESH,
        scratch_types=[
            pltpu.VMEM((NB, K), jnp.int32),
            pltpu.VMEM((RPS,), jnp.float32),
            pltpu.VMEM((K,), jnp.float32),
            pltpu.VMEM_SHARED((NPAD,), jnp.float32),
        ],
    )
    def deg_kernel(dst_hbm, out_hbm, dst_v, zeros_v, ones_v, acc):
        cid = lax.axis_index("c")
        sid = lax.axis_index("s")
        wid = sid * 2 + cid

        @pl.loop(0, RPS, step=16)
        def _(i):
            zeros_v.at[pl.ds(i, 16)][...] = jnp.zeros((16,), jnp.float32)

        @pl.loop(0, K, step=16)
        def _(i):
            ones_v.at[pl.ds(i, 16)][...] = jnp.full((16,), 1.0, jnp.float32)

        pltpu.sync_copy(zeros_v, acc.at[pl.ds(sid * RPS, RPS)])
        pltpu.sync_copy(dst_hbm.at[wid], dst_v)
        plsc.subcore_barrier()

        @pl.loop(0, NB)
        def _(j):
            pltpu.sync_copy(ones_v, acc.at[dst_v.at[j]], add=True)

        plsc.subcore_barrier()
        pltpu.sync_copy(acc.at[pl.ds(sid * RPS, RPS)],
                        out_hbm.at[cid, pl.ds(sid * RPS, RPS)])

    return deg_kernel(dst3)


NBUF = 2
NI = 2 * NBUF

B0 = 156
B1 = 4
NBLK = 16 * (B0 + B1)
NB_MAX = ((max(B0, B1) + NI - 1) // NI) * NI


def _sc_spmm(scaled, sd3, half):
    @functools.partial(
        pl.kernel,
        out_type=jax.ShapeDtypeStruct((NPAD, D), jnp.float32),
        mesh=_MESH1,
        scratch_types=(
            [pltpu.VMEM((2, K), jnp.int32) for _ in range(NI)]
            + [pltpu.VMEM((K, D), jnp.float32) for _ in range(NBUF)]
            + [pltpu.SemaphoreType.DMA for _ in range(NI + NBUF)]
            + [pltpu.VMEM_SHARED((NPAD, D), jnp.float32)]
        ),
    )
    def spmm_kernel(scaled_hbm, sd_hbm, out_hbm, *rest):
        sd = rest[:NI]
        rows = rest[NI:NI + NBUF]
        sem_i = rest[NI + NBUF:2 * NI + NBUF]
        sem_g = rest[2 * NI + NBUF:2 * NI + 2 * NBUF]
        acc = rest[2 * NI + 2 * NBUF]
        sid = lax.axis_index("s")
        nb = jnp.int32(80)
        base = half * 1280 + sid * 80

        @pl.loop(0, K)
        def _(r):
            @pl.loop(0, D, step=16)
            def _(cc):
                rows[0].at[r, pl.ds(cc, 16)][...] = jnp.zeros((16,), jnp.float32)

        @pl.loop(0, RPS, step=K)
        def _(r):
            pltpu.sync_copy(rows[0], acc.at[pl.ds(sid * RPS + r, K)])

        for q in range(NI):
            @pl.when(q < nb)
            def _():
                pltpu.async_copy(sd_hbm.at[base + q], sd[q], sem_i[q])
        plsc.subcore_barrier()
        for b in range(NBUF):
            @pl.when(b < nb)
            def _():
                pltpu.make_async_copy(sd_hbm.at[base + b], sd[b],
                                      sem_i[b]).wait()
                pltpu.async_copy(scaled_hbm.at[sd[b].at[0]], rows[b], sem_g[b])

        nb_up = ((nb + NI - 1) // NI) * NI

        @pl.loop(0, nb_up, step=NI)
        def _(j0):
            for q in range(NI):
                j = j0 + q
                b = q % NBUF

                @pl.when(j < nb)
                def _():
                    pltpu.make_async_copy(scaled_hbm.at[sd[q].at[0]], rows[b],
                                          sem_g[b]).wait()
                    pltpu.sync_copy(rows[b], acc.at[sd[q].at[1]], add=True)

                    @pl.when(j + NBUF < nb)
                    def _():
                        qq = (q + NBUF) % NI
                        pltpu.make_async_copy(sd_hbm.at[base + j + NBUF],
                                              sd[qq], sem_i[qq]).wait()
                        pltpu.async_copy(scaled_hbm.at[sd[qq].at[0]], rows[b],
                                         sem_g[b])

                    @pl.when(j + NI < nb)
                    def _():
                        pltpu.async_copy(sd_hbm.at[base + j + NI], sd[q],
                                         sem_i[q])

        plsc.subcore_barrier()
        pltpu.sync_copy(acc.at[pl.ds(sid * RPS, RPS)],
                        out_hbm.at[pl.ds(sid * RPS, RPS)])

    return spmm_kernel(scaled, sd3)


def _enc_body(x_ref, w1_ref, b1_ref, w2_ref, b2_ref, h_ref):
    h1 = jnp.dot(x_ref[...], w1_ref[...], preferred_element_type=jnp.float32)
    h1 = jnp.maximum(h1 + b1_ref[...], 0.0)
    h2 = jnp.dot(h1, w2_ref[...], preferred_element_type=jnp.float32)
    h_ref[...] = jnp.maximum(h2 + b2_ref[...], 0.0)


def _encoder(x, W1, b1, W2, b2):
    return pl.pallas_call(
        _enc_body,
        grid=(N // RB,),
        in_specs=[
            pl.BlockSpec((RB, D), lambda i: (i, 0)),
            pl.BlockSpec((D, D), lambda i: (0, 0)),
            pl.BlockSpec((1, D), lambda i: (0, 0)),
            pl.BlockSpec((D, D), lambda i: (0, 0)),
            pl.BlockSpec((1, D), lambda i: (0, 0)),
        ],
        out_specs=pl.BlockSpec((RB, D), lambda i: (i, 0)),
        out_shape=jax.ShapeDtypeStruct((N, D), jnp.float32),
    )(x, W1, b1, W2, b2)


def _scale_body(degp_ref, h_ref, dinv_ref, scaled_ref):
    dp = degp_ref[...]
    deg = dp[0] + dp[1]
    dinv = lax.rsqrt(jnp.maximum(deg, 1.0))
    dinv_ref[...] = dinv
    scaled_ref[...] = h_ref[...] * dinv


def _scale(degp, h):
    return pl.pallas_call(
        _scale_body,
        grid=(N // RB,),
        in_specs=[
            pl.BlockSpec((2, RB, 1), lambda i: (0, i, 0)),
            pl.BlockSpec((RB, D), lambda i: (i, 0)),
        ],
        out_specs=[
            pl.BlockSpec((RB, 1), lambda i: (i, 0)),
            pl.BlockSpec((RB, D), lambda i: (i, 0)),
        ],
        out_shape=[
            jax.ShapeDtypeStruct((N, 1), jnp.float32),
            jax.ShapeDtypeStruct((N, D), jnp.float32),
        ],
    )(degp, h)


def _lap_body(aggp_ref, aggq_ref, feat_ref, dinv_ref, u_ref, scaled_ref):
    ap = aggp_ref[...] + aggq_ref[...]
    dinv = dinv_ref[...]
    u = feat_ref[...] - dinv * ap
    u_ref[...] = u
    scaled_ref[...] = dinv * u


def _lap(aggp, aggq, feat, dinv):
    return pl.pallas_call(
        _lap_body,
        grid=(N // RB,),
        in_specs=[
            pl.BlockSpec((RB, D), lambda i: (i, 0)),
            pl.BlockSpec((RB, D), lambda i: (i, 0)),
            pl.BlockSpec((RB, D), lambda i: (i, 0)),
            pl.BlockSpec((RB, 1), lambda i: (i, 0)),
        ],
        out_specs=[
            pl.BlockSpec((RB, D), lambda i: (i, 0)),
            pl.BlockSpec((RB, D), lambda i: (i, 0)),
        ],
        out_shape=[
            jax.ShapeDtypeStruct((N, D), jnp.float32),
            jax.ShapeDtypeStruct((N, D), jnp.float32),
        ],
    )(aggp, aggq, feat, dinv)


def _final_body(u1_ref, aggp_ref, aggq_ref, h_ref, dinv_ref, batch_ref,
                w3_ref, b3_ref, w4_ref, b4_ref, out_ref,
                sums_ref, counts_ref):
    i = pl.program_id(0)
    ap = aggp_ref[...] + aggq_ref[...]
    u1 = u1_ref[...]
    u2 = u1 - dinv_ref[...] * ap
    h = h_ref[...]
    W3 = w3_ref[...]
    Wh = 3.0 * W3[0:D]
    Wa = -3.0 * W3[0:D] + 3.0 * W3[D:2 * D]
    Wb = 0.75 * W3[0:D] - 1.5 * W3[D:2 * D] + 0.75 * W3[2 * D:3 * D]
    y = (jnp.dot(h, Wh, preferred_element_type=jnp.float32)
         + jnp.dot(u1, Wa, preferred_element_type=jnp.float32)
         + jnp.dot(u2, Wb, preferred_element_type=jnp.float32)
         + b3_ref[...])
    y = jnp.maximum(y, 0.0)
    node = jnp.dot(y, w4_ref[...], preferred_element_type=jnp.float32)
    node = node + b4_ref[...]

    gids = lax.broadcasted_iota(jnp.int32, (1, G), 1)
    onehot = (batch_ref[...] == gids).astype(jnp.float32)
    psum = lax.dot_general(onehot, node, (((0,), (0,)), ((), ())),
                           preferred_element_type=jnp.float32)
    pcnt = lax.dot_general(onehot, jnp.ones((onehot.shape[0], 1), jnp.float32),
                           (((0,), (0,)), ((), ())),
                           preferred_element_type=jnp.float32)

    @pl.when(i == 0)
    def _():
        sums_ref[...] = jnp.zeros_like(sums_ref)
        counts_ref[...] = jnp.zeros_like(counts_ref)

    sums_ref[...] += psum
    counts_ref[...] += pcnt

    @pl.when(i == pl.num_programs(0) - 1)
    def _():
        out_ref[...] = sums_ref[...] / jnp.maximum(counts_ref[...], 1.0)


def _final(u1, aggp, aggq, h, dinv, batch2, W3, b3, W4, b4):
    return pl.pallas_call(
        _final_body,
        grid=(N // RB,),
        in_specs=[
            pl.BlockSpec((RB, D), lambda i: (i, 0)),
            pl.BlockSpec((RB, D), lambda i: (i, 0)),
            pl.BlockSpec((RB, D), lambda i: (i, 0)),
            pl.BlockSpec((RB, D), lambda i: (i, 0)),
            pl.BlockSpec((RB, 1), lambda i: (i, 0)),
            pl.BlockSpec((RB, 1), lambda i: (i, 0)),
            pl.BlockSpec((3 * D, D), lambda i: (0, 0)),
            pl.BlockSpec((1, D), lambda i: (0, 0)),
            pl.BlockSpec((D, D), lambda i: (0, 0)),
            pl.BlockSpec((1, D), lambda i: (0, 0)),
        ],
        out_specs=pl.BlockSpec((G, D), lambda i: (0, 0)),
        out_shape=jax.ShapeDtypeStruct((G, D), jnp.float32),
        scratch_shapes=[
            pltpu.VMEM((G, D), jnp.float32),
            pltpu.VMEM((G, 1), jnp.float32),
        ],
    )(u1, aggp, aggq, h, dinv, batch2, W3, b3, W4, b4)


def kernel(x, edge_index, batch, W1, b1, W2, b2, W3, b3, W4, b4):
    src = edge_index[0]
    dst = edge_index[1]
    pad = E_PAD - E
    src_p = jnp.concatenate([src, jnp.zeros((pad,), jnp.int32)])
    dst_p = jnp.concatenate([dst, jnp.full((pad,), N, jnp.int32)])
    dst3 = dst_p.reshape(NW, NB, K)
    sd3 = jnp.stack([src_p.reshape(NBLK, K), dst_p.reshape(NBLK, K)],
                    axis=1)

    h = _encoder(x, W1, b1.reshape(1, D), W2, b2.reshape(1, D))
    degp = _sc_degree(dst3)
    dinv, scaled1 = _scale(degp.reshape(2, NPAD, 1), h)
    agg1a = _sc_spmm(scaled1, sd3, 0)
    agg1b = _sc_spmm(scaled1, sd3, 1)
    u1, scaled2 = _lap(agg1a, agg1b, h, dinv)
    agg2a = _sc_spmm(scaled2, sd3, 0)
    agg2b = _sc_spmm(scaled2, sd3, 1)
    return _final(u1, agg2a, agg2b, h, dinv, batch.reshape(N, 1),
                  W3, b3.reshape(1, D), W4, b4.reshape(1, D))


# speedup vs baseline: 1.4128x; 1.2243x over previous
import functools

import jax
import jax.numpy as jnp
from jax import lax
from jax.experimental import pallas as pl
from jax.experimental.pallas import tpu as pltpu
from jax.experimental.pallas import tpu_sc as plsc

N = 10000
D = 128
E = 320000
G = 128
NW = 32
K = 128
NB = 80
EPW = NB * K
E_PAD = NW * EPW
NPAD = 10240
RPS = NPAD // 16
RB = 2000

_MESH = plsc.VectorSubcoreMesh(core_axis_name="c", subcore_axis_name="s")


def _sc_degree(dst3):
    @functools.partial(
        pl.kernel,
        out_type=jax.ShapeDtypeStruct((2, NPAD), jnp.float32),
        mesh=_MESH,
        scratch_types=[
            pltpu.VMEM((NB, K), jnp.int32),
            pltpu.VMEM((RPS,), jnp.float32),
            pltpu.VMEM((K,), jnp.float32),
            pltpu.VMEM_SHARED((NPAD,), jnp.float32),
        ],
    )
    def deg_kernel(dst_hbm, out_hbm, dst_v, zeros_v, ones_v, acc):
        cid = lax.axis_index("c")
        sid = lax.axis_index("s")
        wid = sid * 2 + cid

        @pl.loop(0, RPS, step=16)
        def _(i):
            zeros_v.at[pl.ds(i, 16)][...] = jnp.zeros((16,), jnp.float32)

        @pl.loop(0, K, step=16)
        def _(i):
            ones_v.at[pl.ds(i, 16)][...] = jnp.full((16,), 1.0, jnp.float32)

        pltpu.sync_copy(zeros_v, acc.at[pl.ds(sid * RPS, RPS)])
        pltpu.sync_copy(dst_hbm.at[wid], dst_v)
        plsc.subcore_barrier()

        @pl.loop(0, NB)
        def _(j):
            pltpu.sync_copy(ones_v, acc.at[dst_v.at[j]], add=True)

        plsc.subcore_barrier()
        pltpu.sync_copy(acc.at[pl.ds(sid * RPS, RPS)],
                        out_hbm.at[cid, pl.ds(sid * RPS, RPS)])

    return deg_kernel(dst3)


NBUF = 2
NI = 2 * NBUF

B0 = 156
B1 = 4
NBLK = 16 * (B0 + B1)
NB_MAX = ((max(B0, B1) + NI - 1) // NI) * NI


def _sc_spmm(scaled, sd3):
    @functools.partial(
        pl.kernel,
        out_type=jax.ShapeDtypeStruct((2, NPAD, D), jnp.float32),
        mesh=_MESH,
        scratch_types=(
            [pltpu.VMEM((2, K), jnp.int32) for _ in range(NI)]
            + [pltpu.VMEM((K, D), jnp.float32) for _ in range(NBUF)]
            + [pltpu.SemaphoreType.DMA for _ in range(NI + NBUF)]
            + [pltpu.VMEM_SHARED((NPAD, D), jnp.float32)]
        ),
    )
    def spmm_kernel(scaled_hbm, sd_hbm, out_hbm, *rest):
        sd = rest[:NI]
        rows = rest[NI:NI + NBUF]
        sem_i = rest[NI + NBUF:2 * NI + NBUF]
        sem_g = rest[2 * NI + NBUF:2 * NI + 2 * NBUF]
        acc = rest[2 * NI + 2 * NBUF]
        cid = lax.axis_index("c")
        sid = lax.axis_index("s")
        nb = jnp.where(cid == 0, B0, B1)
        base = jnp.where(cid == 0, sid * B0, 16 * B0 + sid * B1)

        @pl.loop(0, K)
        def _(r):
            @pl.loop(0, D, step=16)
            def _(cc):
                rows[0].at[r, pl.ds(cc, 16)][...] = jnp.zeros((16,), jnp.float32)

        @pl.loop(0, RPS, step=K)
        def _(r):
            pltpu.sync_copy(rows[0], acc.at[pl.ds(sid * RPS + r, K)])

        for q in range(NI):
            @pl.when(q < nb)
            def _():
                pltpu.async_copy(sd_hbm.at[base + q], sd[q], sem_i[q])
        plsc.subcore_barrier()
        for b in range(NBUF):
            @pl.when(b < nb)
            def _():
                pltpu.make_async_copy(sd_hbm.at[base + b], sd[b],
                                      sem_i[b]).wait()
                pltpu.async_copy(scaled_hbm.at[sd[b].at[0]], rows[b], sem_g[b])

        nb_up = ((nb + NI - 1) // NI) * NI

        @pl.loop(0, nb_up, step=NI)
        def _(j0):
            for q in range(NI):
                j = j0 + q
                b = q % NBUF

                @pl.when(j < nb)
                def _():
                    pltpu.make_async_copy(scaled_hbm.at[sd[q].at[0]], rows[b],
                                          sem_g[b]).wait()
                    pltpu.sync_copy(rows[b], acc.at[sd[q].at[1]], add=True)

                    @pl.when(j + NBUF < nb)
                    def _():
                        qq = (q + NBUF) % NI
                        pltpu.make_async_copy(sd_hbm.at[base + j + NBUF],
                                              sd[qq], sem_i[qq]).wait()
                        pltpu.async_copy(scaled_hbm.at[sd[qq].at[0]], rows[b],
                                         sem_g[b])

                    @pl.when(j + NI < nb)
                    def _():
                        pltpu.async_copy(sd_hbm.at[base + j + NI], sd[q],
                                         sem_i[q])

        plsc.subcore_barrier()
        pltpu.sync_copy(acc.at[pl.ds(sid * RPS, RPS)],
                        out_hbm.at[cid, pl.ds(sid * RPS, RPS)])

    return spmm_kernel(scaled, sd3)


def _enc_body(x_ref, w1_ref, b1_ref, w2_ref, b2_ref, h_ref):
    h1 = jnp.dot(x_ref[...], w1_ref[...], preferred_element_type=jnp.float32)
    h1 = jnp.maximum(h1 + b1_ref[...], 0.0)
    h2 = jnp.dot(h1, w2_ref[...], preferred_element_type=jnp.float32)
    h_ref[...] = jnp.maximum(h2 + b2_ref[...], 0.0)


def _encoder(x, W1, b1, W2, b2):
    return pl.pallas_call(
        _enc_body,
        grid=(N // RB,),
        in_specs=[
            pl.BlockSpec((RB, D), lambda i: (i, 0)),
            pl.BlockSpec((D, D), lambda i: (0, 0)),
            pl.BlockSpec((1, D), lambda i: (0, 0)),
            pl.BlockSpec((D, D), lambda i: (0, 0)),
            pl.BlockSpec((1, D), lambda i: (0, 0)),
        ],
        out_specs=pl.BlockSpec((RB, D), lambda i: (i, 0)),
        out_shape=jax.ShapeDtypeStruct((N, D), jnp.float32),
    )(x, W1, b1, W2, b2)


def _scale_body(degp_ref, h_ref, dinv_ref, scaled_ref):
    dp = degp_ref[...]
    deg = dp[0] + dp[1]
    dinv = lax.rsqrt(jnp.maximum(deg, 1.0))
    dinv_ref[...] = dinv
    scaled_ref[...] = h_ref[...] * dinv


def _scale(degp, h):
    return pl.pallas_call(
        _scale_body,
        grid=(N // RB,),
        in_specs=[
            pl.BlockSpec((2, RB, 1), lambda i: (0, i, 0)),
            pl.BlockSpec((RB, D), lambda i: (i, 0)),
        ],
        out_specs=[
            pl.BlockSpec((RB, 1), lambda i: (i, 0)),
            pl.BlockSpec((RB, D), lambda i: (i, 0)),
        ],
        out_shape=[
            jax.ShapeDtypeStruct((N, 1), jnp.float32),
            jax.ShapeDtypeStruct((N, D), jnp.float32),
        ],
    )(degp, h)


def _lap_body(aggp_ref, feat_ref, dinv_ref, u_ref, scaled_ref):
    ap = aggp_ref[...]
    dinv = dinv_ref[...]
    u = feat_ref[...] - dinv * (ap[0] + ap[1])
    u_ref[...] = u
    scaled_ref[...] = dinv * u


def _lap(aggp, feat, dinv):
    return pl.pallas_call(
        _lap_body,
        grid=(N // RB,),
        in_specs=[
            pl.BlockSpec((2, RB, D), lambda i: (0, i, 0)),
            pl.BlockSpec((RB, D), lambda i: (i, 0)),
            pl.BlockSpec((RB, 1), lambda i: (i, 0)),
        ],
        out_specs=[
            pl.BlockSpec((RB, D), lambda i: (i, 0)),
            pl.BlockSpec((RB, D), lambda i: (i, 0)),
        ],
        out_shape=[
            jax.ShapeDtypeStruct((N, D), jnp.float32),
            jax.ShapeDtypeStruct((N, D), jnp.float32),
        ],
    )(aggp, feat, dinv)


def _final_body(u1_ref, aggp_ref, h_ref, dinv_ref, batch_ref,
                w3_ref, b3_ref, w4_ref, b4_ref, out_ref,
                sums_ref, counts_ref):
    i = pl.program_id(0)
    ap = aggp_ref[...]
    u1 = u1_ref[...]
    u2 = u1 - dinv_ref[...] * (ap[0] + ap[1])
    h = h_ref[...]
    W3 = w3_ref[...]
    Wh = 3.0 * W3[0:D]
    Wa = -3.0 * W3[0:D] + 3.0 * W3[D:2 * D]
    Wb = 0.75 * W3[0:D] - 1.5 * W3[D:2 * D] + 0.75 * W3[2 * D:3 * D]
    y = (jnp.dot(h, Wh, preferred_element_type=jnp.float32)
         + jnp.dot(u1, Wa, preferred_element_type=jnp.float32)
         + jnp.dot(u2, Wb, preferred_element_type=jnp.float32)
         + b3_ref[...])
    y = jnp.maximum(y, 0.0)
    node = jnp.dot(y, w4_ref[...], preferred_element_type=jnp.float32)
    node = node + b4_ref[...]

    gids = lax.broadcasted_iota(jnp.int32, (1, G), 1)
    onehot = (batch_ref[...] == gids).astype(jnp.float32)
    psum = lax.dot_general(onehot, node, (((0,), (0,)), ((), ())),
                           preferred_element_type=jnp.float32)
    pcnt = lax.dot_general(onehot, jnp.ones((onehot.shape[0], 1), jnp.float32),
                           (((0,), (0,)), ((), ())),
                           preferred_element_type=jnp.float32)

    @pl.when(i == 0)
    def _():
        sums_ref[...] = jnp.zeros_like(sums_ref)
        counts_ref[...] = jnp.zeros_like(counts_ref)

    sums_ref[...] += psum
    counts_ref[...] += pcnt

    @pl.when(i == pl.num_programs(0) - 1)
    def _():
        out_ref[...] = sums_ref[...] / jnp.maximum(counts_ref[...], 1.0)


def _final(u1, aggp, h, dinv, batch2, W3, b3, W4, b4):
    return pl.pallas_call(
        _final_body,
        grid=(N // RB,),
        in_specs=[
            pl.BlockSpec((RB, D), lambda i: (i, 0)),
            pl.BlockSpec((2, RB, D), lambda i: (0, i, 0)),
            pl.BlockSpec((RB, D), lambda i: (i, 0)),
            pl.BlockSpec((RB, 1), lambda i: (i, 0)),
            pl.BlockSpec((RB, 1), lambda i: (i, 0)),
            pl.BlockSpec((3 * D, D), lambda i: (0, 0)),
            pl.BlockSpec((1, D), lambda i: (0, 0)),
            pl.BlockSpec((D, D), lambda i: (0, 0)),
            pl.BlockSpec((1, D), lambda i: (0, 0)),
        ],
        out_specs=pl.BlockSpec((G, D), lambda i: (0, 0)),
        out_shape=jax.ShapeDtypeStruct((G, D), jnp.float32),
        scratch_shapes=[
            pltpu.VMEM((G, D), jnp.float32),
            pltpu.VMEM((G, 1), jnp.float32),
        ],
    )(u1, aggp, h, dinv, batch2, W3, b3, W4, b4)


def kernel(x, edge_index, batch, W1, b1, W2, b2, W3, b3, W4, b4):
    src = edge_index[0]
    dst = edge_index[1]
    pad = E_PAD - E
    src_p = jnp.concatenate([src, jnp.zeros((pad,), jnp.int32)])
    dst_p = jnp.concatenate([dst, jnp.full((pad,), N, jnp.int32)])
    dst3 = dst_p.reshape(NW, NB, K)
    sd3 = jnp.stack([src_p.reshape(NBLK, K), dst_p.reshape(NBLK, K)],
                    axis=1)

    h = _encoder(x, W1, b1.reshape(1, D), W2, b2.reshape(1, D))
    degp = _sc_degree(dst3)
    dinv, scaled1 = _scale(degp.reshape(2, NPAD, 1), h)
    agg1 = _sc_spmm(scaled1, sd3)
    u1, scaled2 = _lap(agg1, h, dinv)
    agg2 = _sc_spmm(scaled2, sd3)
    return _final(u1, agg2, h, dinv, batch.reshape(N, 1),
                  W3, b3.reshape(1, D), W4, b4.reshape(1, D))
